# baseline (speedup 1.0000x reference)
import functools

import jax
import jax.numpy as jnp
from jax import lax
from jax.experimental import pallas as pl
from jax.experimental.pallas import tpu as pltpu
from jax.experimental.pallas import tpu_sc as plsc

B, S, D = 4, 8192, 768
NC, NS = 2, 16
NW = NC * NS
SEQ_PER_W = S // NW
R = 32
NCH = SEQ_PER_W // R
NSTEP = NCH * B
LANES = 16
GPR = D // LANES


def _sc_broadcast_add(x, emb):
    mesh = plsc.VectorSubcoreMesh(core_axis_name="c", subcore_axis_name="s")

    @functools.partial(
        pl.kernel,
        mesh=mesh,
        out_type=jax.ShapeDtypeStruct((B, S, D), jnp.float32),
        compiler_params=pltpu.CompilerParams(use_tc_tiling_on_sc=True),
        scratch_types=[
            pltpu.VMEM((R, D), jnp.float32),
            pltpu.VMEM((R, D), jnp.float32),
            pltpu.VMEM((R, D), jnp.float32),
            pltpu.VMEM((R, D), jnp.float32),
            pltpu.VMEM((R, D), jnp.float32),
            pltpu.SemaphoreType.DMA,
            pltpu.SemaphoreType.DMA,
            pltpu.SemaphoreType.DMA,
            pltpu.SemaphoreType.DMA,
            pltpu.SemaphoreType.DMA,
            pltpu.SemaphoreType.DMA,
            pltpu.SemaphoreType.DMA,
            pltpu.SemaphoreType.DMA,
        ],
    )
    def k(x_hbm, e_hbm, o_hbm, xb0, xb1, xb2, eb0, eb1,
          si0, si1, si2, so0, so1, so2, se0, se1):
        xb = [xb0, xb1, xb2]
        eb = [eb0, eb1]
        si = [si0, si1, si2]
        so = [so0, so1, so2]
        se = [se0, se1]
        wid = lax.axis_index("s") * NC + lax.axis_index("c")
        row0 = wid * SEQ_PER_W

        def e_src(ci):
            return e_hbm.at[pl.ds(row0 + ci * R, R)]

        def x_src(t):
            ci, b = divmod(t, B)
            return x_hbm.at[b, pl.ds(row0 + ci * R, R)]

        def o_dst(t):
            ci, b = divmod(t, B)
            return o_hbm.at[b, pl.ds(row0 + ci * R, R)]

        pltpu.async_copy(e_src(0), eb[0], se[0])
        pltpu.async_copy(e_src(1), eb[1], se[1])
        for t in range(3):
            pltpu.async_copy(x_src(t), xb[t], si[t])

        for t in range(NSTEP):
            ci, b = divmod(t, B)
            p = t % 3
            ep = ci % 2
            if b == 0:
                pltpu.make_async_copy(e_src(ci), eb[ep], se[ep]).wait()
            pltpu.make_async_copy(x_src(t), xb[p], si[p]).wait()

            def row_body(r, c2, _p=p, _ep=ep):
                def col_body(j, c3, _r=r):
                    for u in range(8):
                        sl = pl.ds(j * 128 + u * LANES, LANES)
                        xb[_p][_r, sl] = xb[_p][_r, sl] + eb[_ep][_r, sl]
                    return c3

                lax.fori_loop(0, GPR // 8, col_body, 0)
                return c2

            lax.fori_loop(0, R, row_body, 0)

            pltpu.async_copy(xb[p], o_dst(t), so[p])

            nt = t + 2
            if 3 <= nt < NSTEP:
                q = nt % 3
                pltpu.make_async_copy(xb[q], o_dst(t - 1), so[q]).wait()
                pltpu.async_copy(x_src(nt), xb[q], si[q])
            if b == B - 1 and ci + 2 < NCH:
                pltpu.async_copy(e_src(ci + 2), eb[ep], se[ep])

        lt = NSTEP - 1
        pltpu.make_async_copy(xb[lt % 3], o_dst(lt), so[lt % 3]).wait()

    return k(x, emb)


def kernel(x, emb_table):
    return _sc_broadcast_add(x, emb_table)


# speedup vs baseline: 3.2246x; 3.2246x over previous
import functools

import jax
import jax.numpy as jnp
from jax.experimental import pallas as pl
from jax.experimental.pallas import tpu as pltpu

B, S, D = 4, 8192, 768
BS = 512


def _tc_add(x, emb):
    def body(x_ref, e_ref, o_ref):
        o_ref[...] = x_ref[...] + e_ref[...][None]

    return pl.pallas_call(
        body,
        grid=(B, S // BS),
        in_specs=[
            pl.BlockSpec((1, BS, D), lambda b, i: (b, i, 0)),
            pl.BlockSpec((BS, D), lambda b, i: (i, 0)),
        ],
        out_specs=pl.BlockSpec((1, BS, D), lambda b, i: (b, i, 0)),
        out_shape=jax.ShapeDtypeStruct((B, S, D), jnp.float32),
    )(x, emb)


def kernel(x, emb_table):
    return _tc_add(x, emb_table)
